# baseline (speedup 1.0000x reference)
import functools

import jax
import jax.numpy as jnp
from jax import lax
from jax.experimental import pallas as pl
from jax.experimental.pallas import tpu as pltpu

NE = 10
LANES = 128
SUB = 8


def _tree_sum(acc):
    t = acc[0:4, :] + acc[4:8, :]
    t = t[0:2, :] + t[2:4, :]
    return t[0:1, :] + t[1:2, :]


def _tree_max(acc):
    t = jnp.maximum(acc[0:4, :], acc[4:8, :])
    t = jnp.maximum(t[0:2, :], t[2:4, :])
    return jnp.maximum(t[0:1, :], t[1:2, :])


def _gl_kernel(n, posF_ref, posT_ref, x2F_ref, x2T_ref, uT_ref, inv_ref,
               idx_ref, attr_ref, wT_ref, eT_ref, psT_ref):
    nv = n // SUB
    ntile = n // 128
    j0 = pl.program_id(1)

    pall = posF_ref[0]
    prow = posT_ref[0]
    dotT = lax.dot_general(pall, prow, (((1,), (0,)), ((), ())),
                           precision='default',
                           preferred_element_type=jnp.float32)
    sqT = (x2F_ref[0] + x2T_ref[0]) - 2.0 * dotT
    cdT = jnp.sqrt(jnp.maximum(sqT, 1e-12))
    w = jnp.exp(-(cdT * inv_ref[0, 0]))
    jcol = lax.broadcasted_iota(jnp.int32, (n, LANES), 0)
    rglob = j0 * LANES + lax.broadcasted_iota(jnp.int32, (n, LANES), 1)
    wT_ref[...] = jnp.where(jcol == rglob, 0.0, w)

    def _mx(v, acc):
        return jnp.maximum(acc, wT_ref[pl.ds(SUB * v, SUB), :])
    mx = lax.fori_loop(1, nv, _mx, wT_ref[0:SUB, :])
    m = _tree_max(mx)

    eT_ref[...] = jnp.exp(wT_ref[...] - m)

    def _sm(v, acc):
        return acc + eT_ref[pl.ds(SUB * v, SUB), :]
    sacc = lax.fori_loop(1, nv, _sm, eT_ref[0:SUB, :])
    s = _tree_sum(sacc)

    psT_ref[...] = eT_ref[...] / s

    def _scan(i, _):
        for t in range(ntile):
            b = t * 128
            psT_ref[pl.ds(b + i, 1), :] = (psT_ref[pl.ds(b + i - 1, 1), :]
                                           + psT_ref[pl.ds(b + i, 1), :])
        return 0
    lax.fori_loop(1, 128, _scan, 0)

    off = psT_ref[127:128, :]
    for t in range(1, ntile):
        b = t * 128
        tot = psT_ref[b + 127:b + 128, :]
        psT_ref[b:b + 128, :] = psT_ref[b:b + 128, :] + off
        off = off + tot

    iota8 = lax.broadcasted_iota(jnp.int32, (SUB, LANES), 0)
    for k in range(NE):
        ue = uT_ref[0, k:k + 1, :]

        def _cnt(v, acc):
            return acc + jnp.where(psT_ref[pl.ds(SUB * v, SUB), :] < ue,
                                   1.0, 0.0)
        cacc = lax.fori_loop(0, nv, _cnt, jnp.zeros((SUB, LANES), jnp.float32))
        cnt = _tree_sum(cacc)
        idxv = jnp.clip(cnt.astype(jnp.int32), 0, n - 1)

        def _att(v, acc):
            sel = jnp.where(SUB * v + iota8 == idxv,
                            wT_ref[pl.ds(SUB * v, SUB), :], 0.0)
            return acc + sel
        aacc = lax.fori_loop(0, nv, _att, jnp.zeros((SUB, LANES), jnp.float32))
        attrv = _tree_sum(aacc)

        idx_ref[0, k:k + 1, :] = idxv
        attr_ref[0, k:k + 1, :] = attrv


def kernel(pos, batch, eps):
    B = 4
    N, d = pos.shape
    n = N // B
    pos_b = pos.reshape(B, n, d)
    posT = jnp.transpose(pos_b, (0, 2, 1))
    p0, p1, p2 = pos_b[..., 0], pos_b[..., 1], pos_b[..., 2]
    x2 = (p0 * p0 + p2 * p2) + p1 * p1
    x2F = x2[:, :, None]
    x2T = x2[:, None, :]
    inv = (1.0 / (eps[0] ** 2)).reshape(1, 1).astype(jnp.float32)
    u = jax.random.uniform(jax.random.key(42), (B, n, NE), dtype=jnp.float32)
    uT = jnp.transpose(u, (0, 2, 1))

    nb = n // LANES
    grid = (B, nb)
    idxT, attrT = pl.pallas_call(
        functools.partial(_gl_kernel, n),
        grid=grid,
        in_specs=[
            pl.BlockSpec((1, n, 3), lambda b, j: (b, 0, 0)),
            pl.BlockSpec((1, 3, LANES), lambda b, j: (b, 0, j)),
            pl.BlockSpec((1, n, 1), lambda b, j: (b, 0, 0)),
            pl.BlockSpec((1, 1, LANES), lambda b, j: (b, 0, j)),
            pl.BlockSpec((1, NE, LANES), lambda b, j: (b, 0, j)),
            pl.BlockSpec((1, 1), lambda b, j: (0, 0)),
        ],
        out_specs=[
            pl.BlockSpec((1, NE, LANES), lambda b, j: (b, 0, j)),
            pl.BlockSpec((1, NE, LANES), lambda b, j: (b, 0, j)),
        ],
        out_shape=[
            jax.ShapeDtypeStruct((B, NE, n), jnp.int32),
            jax.ShapeDtypeStruct((B, NE, n), jnp.float32),
        ],
        scratch_shapes=[
            pltpu.VMEM((n, LANES), jnp.float32),
            pltpu.VMEM((n, LANES), jnp.float32),
            pltpu.VMEM((n, LANES), jnp.float32),
        ],
    )(pos_b, posT, x2F, x2T, uT, inv)

    idx_target = jnp.transpose(idxT, (0, 2, 1))
    edge_attr = jnp.transpose(attrT, (0, 2, 1)).reshape(-1)
    offsets = (batch.reshape(B, n)[:, 0] * n)[:, None, None]
    idx_src = jnp.broadcast_to(jnp.arange(n)[:, None], (n, NE))
    ei_src = (idx_src[None, :, :] + offsets).reshape(-1)
    ei_dst = (idx_target + offsets).reshape(-1)
    edge_index = jnp.stack([ei_src, ei_dst], axis=0)
    return edge_index, edge_attr


# speedup vs baseline: 7.0045x; 7.0045x over previous
import functools

import jax
import jax.numpy as jnp
from jax import lax
from jax.experimental import pallas as pl
from jax.experimental.pallas import tpu as pltpu

NE = 10
LANES = 512
SUB = 8


def _gl_kernel(n, posF_ref, posT_ref, x2F_ref, x2T_ref, uT_ref, inv_ref,
               idx_ref, attr_ref, w_ref, ps_ref):
    ntile = n // 128
    j0 = pl.program_id(1)

    pall = posF_ref[0]
    prow = posT_ref[0]
    dotT = lax.dot_general(pall, prow, (((1,), (0,)), ((), ())),
                           precision='default',
                           preferred_element_type=jnp.float32)
    sqT = (x2F_ref[0] + x2T_ref[0]) - 2.0 * dotT
    cdT = jnp.sqrt(jnp.maximum(sqT, 1e-12))
    w2 = jnp.exp(-(cdT * inv_ref[0, 0]))
    jcol = lax.broadcasted_iota(jnp.int32, (n, LANES), 0)
    rglob = j0 * LANES + lax.broadcasted_iota(jnp.int32, (n, LANES), 1)
    w_ref[...] = jnp.where(jcol == rglob, 0.0, w2).reshape(ntile, 128, LANES)

    m = jnp.max(jnp.max(w_ref[...], axis=0), axis=0)[None, None, :]

    ps_ref[...] = jnp.exp(w_ref[...] - m)

    def _sm(v, acc):
        return acc + ps_ref[v // 16, pl.ds(SUB * (v % 16), SUB), :]
    sacc = lax.fori_loop(1, ntile * 16, _sm, ps_ref[0, 0:SUB, :])
    t = sacc[0:4, :] + sacc[4:8, :]
    t = t[0:2, :] + t[2:4, :]
    s = (t[0:1, :] + t[1:2, :])[None]

    ps_ref[...] = ps_ref[...] / s

    def _scan(i, _):
        ps_ref[:, pl.ds(i, 1), :] = (ps_ref[:, pl.ds(i - 1, 1), :]
                                     + ps_ref[:, pl.ds(i, 1), :])
        return 0
    lax.fori_loop(1, 128, _scan, 0)

    lasts = ps_ref[:, 127:128, :]
    offs = [jnp.zeros((1, 1, LANES), jnp.float32)]
    acc = lasts[0:1]
    for tt in range(1, ntile):
        offs.append(acc)
        if tt < ntile - 1:
            acc = acc + lasts[tt:tt + 1]
    ps_ref[...] = ps_ref[...] + jnp.concatenate(offs, axis=0)

    coliota = (128 * lax.broadcasted_iota(jnp.int32, (ntile, 128, LANES), 0)
               + lax.broadcasted_iota(jnp.int32, (ntile, 128, LANES), 1))
    psv = ps_ref[...]
    wv = w_ref[...]
    for k in range(NE):
        ue = uT_ref[0, k:k + 1, :][None]
        cnt = jnp.sum(jnp.where(psv < ue, 1.0, 0.0), axis=(0, 1))
        idxv = jnp.clip(cnt.astype(jnp.int32), 0, n - 1)[None, :]
        attrv = jnp.sum(jnp.where(coliota == idxv[None], wv, 0.0), axis=(0, 1))
        idx_ref[0, k:k + 1, :] = idxv
        attr_ref[0, k:k + 1, :] = attrv[None, :]


def kernel(pos, batch, eps):
    B = 4
    N, d = pos.shape
    n = N // B
    pos_b = pos.reshape(B, n, d)
    posT = jnp.transpose(pos_b, (0, 2, 1))
    p0, p1, p2 = pos_b[..., 0], pos_b[..., 1], pos_b[..., 2]
    x2 = (p0 * p0 + p2 * p2) + p1 * p1
    x2F = x2[:, :, None]
    x2T = x2[:, None, :]
    inv = (1.0 / (eps[0] ** 2)).reshape(1, 1).astype(jnp.float32)
    u = jax.random.uniform(jax.random.key(42), (B, n, NE), dtype=jnp.float32)
    uT = jnp.transpose(u, (0, 2, 1))

    nb = n // LANES
    grid = (B, nb)
    idxT, attrT = pl.pallas_call(
        functools.partial(_gl_kernel, n),
        grid=grid,
        in_specs=[
            pl.BlockSpec((1, n, 3), lambda b, j: (b, 0, 0)),
            pl.BlockSpec((1, 3, LANES), lambda b, j: (b, 0, j)),
            pl.BlockSpec((1, n, 1), lambda b, j: (b, 0, 0)),
            pl.BlockSpec((1, 1, LANES), lambda b, j: (b, 0, j)),
            pl.BlockSpec((1, NE, LANES), lambda b, j: (b, 0, j)),
            pl.BlockSpec((1, 1), lambda b, j: (0, 0)),
        ],
        out_specs=[
            pl.BlockSpec((1, NE, LANES), lambda b, j: (b, 0, j)),
            pl.BlockSpec((1, NE, LANES), lambda b, j: (b, 0, j)),
        ],
        out_shape=[
            jax.ShapeDtypeStruct((B, NE, n), jnp.int32),
            jax.ShapeDtypeStruct((B, NE, n), jnp.float32),
        ],
        scratch_shapes=[
            pltpu.VMEM((n // 128, 128, LANES), jnp.float32),
            pltpu.VMEM((n // 128, 128, LANES), jnp.float32),
        ],
    )(pos_b, posT, x2F, x2T, uT, inv)

    idx_target = jnp.transpose(idxT, (0, 2, 1))
    edge_attr = jnp.transpose(attrT, (0, 2, 1)).reshape(-1)
    offsets = (batch.reshape(B, n)[:, 0] * n)[:, None, None]
    idx_src = jnp.broadcast_to(jnp.arange(n)[:, None], (n, NE))
    ei_src = (idx_src[None, :, :] + offsets).reshape(-1)
    ei_dst = (idx_target + offsets).reshape(-1)
    edge_index = jnp.stack([ei_src, ei_dst], axis=0)
    return edge_index, edge_attr
